# baseline (speedup 1.0000x reference)
import functools

import jax
import jax.numpy as jnp
from jax import lax
from jax.experimental import pallas as pl
from jax.experimental.pallas import tpu as pltpu
from jax.experimental.pallas import tpu_sc as plsc

D_MODEL = 64
SCALE = 8.0

NC = 2
NS = 16
NW = NC * NS
LANES = 16

SEQ = 16384
NPOS = 50
C = 128
NCHUNK = (SEQ * NPOS) // (NW * C)
NBUF = 4
VROWS = 2000000


@functools.partial(
    pl.kernel,
    out_type=jax.ShapeDtypeStruct((NPOS, 8, SEQ // C, 8, C), jnp.float32),
    mesh=plsc.VectorSubcoreMesh(core_axis_name="c", subcore_axis_name="s"),
    compiler_params=pltpu.CompilerParams(
        use_tc_tiling_on_sc=False, needs_layout_passes=False),
    scratch_types=[
        pltpu.VMEM((NCHUNK, C), jnp.int32),
        pltpu.VMEM((NBUF, C, D_MODEL), jnp.float32),
        pltpu.VMEM((NBUF, 8, 8, C), jnp.float32),
        pltpu.SemaphoreType.DMA((NBUF,)),
        pltpu.SemaphoreType.DMA((NBUF,)),
    ],
)
def _emb_lookup(table_hbm, x_hbm, out_hbm, idx_v, rows_v, obuf_v, gsem, ssem):
    wid = lax.axis_index("s") * NC + lax.axis_index("c")

    pltpu.sync_copy(x_hbm.at[wid], idx_v)

    iota = lax.iota(jnp.int32, LANES)
    row_idx = [iota + k * LANES for k in range(C // LANES)]
    NBLK = C // LANES

    def chunk_dst(c):
        g = wid * NCHUNK + c
        t = g // (SEQ // C)
        st = g % (SEQ // C)
        return out_hbm.at[t, :, st]

    def transpose_scale(b):
        rows = rows_v.at[b]
        obuf = obuf_v.at[b]

        @plsc.parallel_loop(0, (D_MODEL // LANES) * LANES, unroll=4)
        def dbody(m):
            j = m & (LANES - 1)
            d0 = (m >> 4) << 4
            dvec = d0 + ((iota + j) & (LANES - 1))
            dt = dvec >> 3
            dl = dvec & 7
            for k in range(NBLK):
                v = plsc.load_gather(rows, [row_idx[k], dvec])
                plsc.store_scatter(obuf, [dt, dl, row_idx[k]], v * SCALE)

    def start_gather(c, b):
        pltpu.async_copy(table_hbm.at[idx_v.at[c]], rows_v.at[b], gsem.at[b])

    def wait_gather(c, b):
        pltpu.make_async_copy(
            table_hbm.at[idx_v.at[c]], rows_v.at[b], gsem.at[b]).wait()

    def start_store(c, b):
        pltpu.async_copy(obuf_v.at[b], chunk_dst(c), ssem.at[b])

    def wait_store(c, b):
        pltpu.make_async_copy(obuf_v.at[b], chunk_dst(c), ssem.at[b]).wait()

    for b in range(NBUF):
        start_gather(b, b)

    for b in range(NBUF):
        wait_gather(b, b)
        transpose_scale(b)
        start_store(b, b)
        start_gather(b + NBUF, b)

    def outer(c0, carry):
        for b in range(NBUF):
            c = c0 * NBUF + b
            wait_gather(c, b)
            wait_store(c - NBUF, b)
            transpose_scale(b)
            start_store(c, b)
            start_gather(c + NBUF, b)
        return carry
    lax.fori_loop(1, NCHUNK // NBUF - 1, outer, 0)

    for b in range(NBUF):
        c = NCHUNK - NBUF + b
        wait_gather(c, b)
        wait_store(c - NBUF, b)
        transpose_scale(b)
        start_store(c, b)
    for b in range(NBUF):
        wait_store(NCHUNK - NBUF + b, b)


def kernel(x, table):
    tpad = jnp.broadcast_to(
        table[:, None, :], (VROWS // 2, 2, D_MODEL)).reshape(VROWS, D_MODEL)
    xt = (x.astype(jnp.int32) * 2).T.reshape(NW, NCHUNK, C)
    out5 = _emb_lookup(tpad, xt)
    return out5.transpose(2, 4, 0, 1, 3).reshape(SEQ, NPOS, D_MODEL)


# speedup vs baseline: 2.1673x; 2.1673x over previous
import functools

import jax
import jax.numpy as jnp
from jax import lax
from jax.experimental import pallas as pl
from jax.experimental.pallas import tpu as pltpu
from jax.experimental.pallas import tpu_sc as plsc

D_MODEL = 64
SCALE = 8.0

NC = 2
NS = 16
NW = NC * NS
LANES = 16

SEQ = 16384
NPOS = 50
C = 128
NCHUNK = (SEQ * NPOS) // (NW * C)
NBUF = 4
VROWS = 2000000


@functools.partial(
    pl.kernel,
    out_type=jax.ShapeDtypeStruct((NPOS, 8, SEQ // C, 8, C), jnp.float32),
    mesh=plsc.VectorSubcoreMesh(core_axis_name="c", subcore_axis_name="s"),
    compiler_params=pltpu.CompilerParams(
        use_tc_tiling_on_sc=False, needs_layout_passes=False),
    scratch_types=[
        pltpu.VMEM((NCHUNK, C), jnp.int32),
        pltpu.VMEM((NBUF, C, D_MODEL), jnp.float32),
        pltpu.VMEM((NBUF, 8, 8, C), jnp.float32),
        pltpu.SemaphoreType.DMA((NBUF,)),
        pltpu.SemaphoreType.DMA((NBUF,)),
    ],
)
def _emb_lookup(table_hbm, x_hbm, out_hbm, idx_v, rows_v, obuf_v, gsem, ssem):
    wid = lax.axis_index("s") * NC + lax.axis_index("c")

    pltpu.sync_copy(x_hbm.at[wid], idx_v)

    iota = lax.iota(jnp.int32, LANES)
    row_idx = [iota + k * LANES for k in range(C // LANES)]
    NBLK = C // LANES

    def chunk_dst(c):
        g = wid * NCHUNK + c
        t = g // (SEQ // C)
        st = g % (SEQ // C)
        return out_hbm.at[t, :, st]

    def transpose_scale(b):
        rows = rows_v.at[b]
        obuf = obuf_v.at[b]

        @plsc.parallel_loop(0, (D_MODEL // LANES) * LANES, unroll=4)
        def dbody(m):
            j = m & (LANES - 1)
            d0 = (m >> 4) << 4
            dvec = d0 + ((iota + j) & (LANES - 1))
            dt = dvec >> 3
            dl = dvec & 7
            for k in range(NBLK):
                v = plsc.load_gather(rows, [row_idx[k], dvec])
                plsc.store_scatter(obuf, [dt, dl, row_idx[k]], v * SCALE)

    def start_gather(c, b):
        pltpu.async_copy(table_hbm.at[idx_v.at[c]], rows_v.at[b], gsem.at[b])

    def wait_gather(c, b):
        pltpu.make_async_copy(
            table_hbm.at[idx_v.at[c]], rows_v.at[b], gsem.at[b]).wait()

    def start_store(c, b):
        pltpu.async_copy(obuf_v.at[b], chunk_dst(c), ssem.at[b])

    def wait_store(c, b):
        pltpu.make_async_copy(obuf_v.at[b], chunk_dst(c), ssem.at[b]).wait()

    for b in range(NBUF):
        start_gather(b, b)

    for b in range(NBUF):
        wait_gather(b, b)
        transpose_scale(b)
        start_store(b, b)
        start_gather(b + NBUF, b)

    def outer(c0, carry):
        for b in range(NBUF):
            c = c0 * NBUF + b
            wait_gather(c, b)
            wait_store(c - NBUF, b)
            transpose_scale(b)
            start_store(c, b)
            start_gather(c + NBUF, b)
        return carry
    lax.fori_loop(1, NCHUNK // NBUF - 1, outer, 0)

    for b in range(NBUF):
        c = NCHUNK - NBUF + b
        wait_gather(c, b)
        wait_store(c - NBUF, b)
        transpose_scale(b)
        start_store(c, b)
    for b in range(NBUF):
        wait_store(NCHUNK - NBUF + b, b)


def kernel(x, table):
    tpad = jnp.concatenate(
        [table, jnp.zeros((VROWS // 2, D_MODEL), jnp.float32)],
        axis=1).reshape(VROWS, D_MODEL)
    xt = (x.astype(jnp.int32) * 2).T.reshape(NW, NCHUNK, C)
    out5 = _emb_lookup(tpad, xt)
    return out5.transpose(2, 4, 0, 1, 3).reshape(SEQ, NPOS, D_MODEL)
